# baseline (speedup 1.0000x reference)
import functools

import jax
import jax.numpy as jnp
from jax import lax
from jax.experimental import pallas as pl
from jax.experimental.pallas import tpu as pltpu
from jax.experimental.pallas import tpu_sc as plsc

_NC = 2
_NS = 16
_NW = _NC * _NS
_CH = 128
_UPAD = 100096


def _add_rows(a_v, b_v, rows, width):
    def body(r, carry):
        for k in range(width // 16):
            s = pl.ds(16 * k, 16)
            a_v[r, s] = a_v[r, s] + b_v[r, s]
        return carry

    lax.fori_loop(0, rows, body, 0)


def _sc_gamma_body(F, bpw,
                   user_h, item_h, Gu_h, DGu_h, Gi_h, DGi_h, Bi_h,
                   gu_out, gi_out, be_out, dg_out,
                   idx_v, a_v, b_v, g_v, be_v, dg_v, sem, sem2):
    cid = lax.axis_index("c")
    sid = lax.axis_index("s")
    wid = sid * _NC + cid
    for c in range(bpw // _CH):
        base = wid * bpw + c * _CH
        pltpu.sync_copy(user_h.at[pl.ds(base, _CH)], idx_v)
        pltpu.async_copy(Gu_h.at[idx_v], g_v, sem).wait()
        pltpu.async_copy(DGu_h.at[idx_v], b_v, sem).wait()
        _add_rows(g_v, b_v, _CH, F)
        pltpu.sync_copy(g_v, gu_out.at[pl.ds(base, _CH)])

        pltpu.sync_copy(item_h.at[pl.ds(base, _CH)], idx_v)
        pltpu.async_copy(Bi_h.at[idx_v], be_v, sem2)
        pltpu.async_copy(Gi_h.at[idx_v], a_v, sem).wait()
        pltpu.async_copy(DGi_h.at[idx_v], b_v, sem).wait()
        _add_rows(a_v, b_v, _CH, F)
        pltpu.sync_copy(a_v, gi_out.at[pl.ds(base, _CH)])

        lane0 = lax.iota(jnp.int32, 16) == 0

        def dgbody(r, carry):
            acc = g_v[r, pl.ds(0, 16)] * a_v[r, pl.ds(0, 16)]
            for k in range(1, F // 16):
                s = pl.ds(16 * k, 16)
                acc = acc + g_v[r, s] * a_v[r, s]
            tot = jnp.sum(acc)
            plsc.store_scatter(dg_v, [jnp.full((16,), r, jnp.int32)],
                               jnp.full((16,), tot, jnp.float32), mask=lane0)
            return carry

        lax.fori_loop(0, _CH, dgbody, 0)
        pltpu.sync_copy(dg_v, dg_out.at[pl.ds(base, _CH)])

        pltpu.make_async_copy(Bi_h.at[pl.ds(0, _CH)], be_v, sem2).wait()
        pltpu.sync_copy(be_v, be_out.at[pl.ds(base, _CH)])


def _sc_gamma(user, item, Gu, DGu, Gi, DGi, Bi, interpret=False):
    B = user.shape[0]
    F = Gu.shape[1]
    bpw = B // _NW
    mesh = plsc.VectorSubcoreMesh(
        core_axis_name="c", subcore_axis_name="s",
        num_cores=_NC, num_subcores=_NS)
    f = pl.kernel(
        functools.partial(_sc_gamma_body, F, bpw),
        out_type=(
            jax.ShapeDtypeStruct((B, F), jnp.float32),
            jax.ShapeDtypeStruct((B, F), jnp.float32),
            jax.ShapeDtypeStruct((B,), jnp.float32),
            jax.ShapeDtypeStruct((B,), jnp.float32),
        ),
        mesh=mesh,
        scratch_types=(
            pltpu.VMEM((_CH,), jnp.int32),
            pltpu.VMEM((_CH, F), jnp.float32),
            pltpu.VMEM((_CH, F), jnp.float32),
            pltpu.VMEM((_CH, F), jnp.float32),
            pltpu.VMEM((_CH,), jnp.float32),
            pltpu.VMEM((_CH,), jnp.float32),
            pltpu.SemaphoreType.DMA,
            pltpu.SemaphoreType.DMA,
        ),
        compiler_params=pltpu.CompilerParams(
            use_tc_tiling_on_sc=False, needs_layout_passes=False),
        interpret=interpret,
    )
    return f(user, item, Gu, DGu, Gi, DGi, Bi)


def _sc_theta_body(FD, bpw, user_h, TuT_h, thT_out,
                   idx_v, idx2_v, tht_v, sem2):
    cid = lax.axis_index("c")
    sid = lax.axis_index("s")
    wid = sid * _NC + cid
    for c in range(bpw // _CH):
        base = wid * bpw + c * _CH
        pltpu.sync_copy(user_h.at[pl.ds(base, _CH)], idx_v)

        def jbody(j, carry):
            for k in range(_CH // 16):
                s = pl.ds(16 * k, 16)
                idx2_v[j, s] = idx_v[s] + j * _UPAD
            pltpu.async_copy(TuT_h.at[idx2_v.at[j]], tht_v.at[j], sem2)
            return carry

        lax.fori_loop(0, FD, jbody, 0)
        pltpu.make_async_copy(
            thT_out.at[:, pl.ds(0, _CH)], tht_v, sem2).wait()
        pltpu.sync_copy(tht_v, thT_out.at[:, pl.ds(base, _CH)])


def _sc_theta(user, TuT_flat, FD, interpret=False):
    B = user.shape[0]
    bpw = B // _NW
    mesh = plsc.VectorSubcoreMesh(
        core_axis_name="c", subcore_axis_name="s",
        num_cores=_NC, num_subcores=_NS)
    f = pl.kernel(
        functools.partial(_sc_theta_body, FD, bpw),
        out_type=jax.ShapeDtypeStruct((FD, B), jnp.float32),
        mesh=mesh,
        scratch_types=(
            pltpu.VMEM((_CH,), jnp.int32),
            pltpu.VMEM((FD, _CH), jnp.int32),
            pltpu.VMEM((FD, _CH), jnp.float32),
            pltpu.SemaphoreType.DMA,
        ),
        compiler_params=pltpu.CompilerParams(
            use_tc_tiling_on_sc=False, needs_layout_passes=False),
        interpret=interpret,
    )
    return f(user, TuT_flat)


def _tc_pad_body(U, t_ref, o_ref):
    for r in range(8):
        o_ref[pl.ds(r * _UPAD, U)] = t_ref[r, :]
        o_ref[pl.ds(r * _UPAD + U, _UPAD - U)] = jnp.zeros(
            (_UPAD - U,), jnp.float32)


def _tc_pad(TuT, interpret=False):
    FD, U = TuT.shape
    return pl.pallas_call(
        functools.partial(_tc_pad_body, U),
        grid=(FD // 8,),
        in_specs=[pl.BlockSpec((8, U), lambda i: (i, 0))],
        out_specs=pl.BlockSpec((8 * _UPAD,), lambda i: (i,)),
        out_shape=jax.ShapeDtypeStruct((FD * _UPAD,), jnp.float32),
        interpret=interpret,
    )(TuT)


def _tc_project_body(f_ref, E_ref, Bp_ref, projT_ref, fbp_ref, fout_ref):
    f = f_ref[...]
    proj = jnp.dot(f, E_ref[...], preferred_element_type=jnp.float32)
    projT_ref[...] = proj.T
    fbp_ref[...] = jnp.dot(f, Bp_ref[...], preferred_element_type=jnp.float32)
    fout_ref[...] = f


def _tc_project(feature_i, E, Bp, interpret=False):
    B, NIF = feature_i.shape
    FD = E.shape[1]
    BB = 1024
    return pl.pallas_call(
        _tc_project_body,
        grid=(B // BB,),
        in_specs=[
            pl.BlockSpec((BB, NIF), lambda i: (i, 0)),
            pl.BlockSpec((NIF, FD), lambda i: (0, 0)),
            pl.BlockSpec((NIF, 1), lambda i: (0, 0)),
        ],
        out_specs=[
            pl.BlockSpec((FD, BB), lambda i: (0, i)),
            pl.BlockSpec((BB, 1), lambda i: (i, 0)),
            pl.BlockSpec((BB, NIF), lambda i: (i, 0)),
        ],
        out_shape=[
            jax.ShapeDtypeStruct((FD, B), jnp.float32),
            jax.ShapeDtypeStruct((B, 1), jnp.float32),
            jax.ShapeDtypeStruct((B, NIF), jnp.float32),
        ],
        interpret=interpret,
    )(feature_i, E, Bp)


def _tc_combine_body(thT_ref, projT_ref, dg_ref, fbp_ref, be_ref, xui_ref):
    dtT = jnp.sum(thT_ref[...] * projT_ref[...], axis=0)
    xui_ref[...] = be_ref[...] + dg_ref[...] + dtT + fbp_ref[...][:, 0]


def _tc_combine(thT, projT, dg, fbp, beta_i, interpret=False):
    FD, B = thT.shape
    BB = 4096
    return pl.pallas_call(
        _tc_combine_body,
        grid=(B // BB,),
        in_specs=[
            pl.BlockSpec((FD, BB), lambda i: (0, i)),
            pl.BlockSpec((FD, BB), lambda i: (0, i)),
            pl.BlockSpec((BB,), lambda i: (i,)),
            pl.BlockSpec((BB, 1), lambda i: (i, 0)),
            pl.BlockSpec((BB,), lambda i: (i,)),
        ],
        out_specs=pl.BlockSpec((BB,), lambda i: (i,)),
        out_shape=jax.ShapeDtypeStruct((B,), jnp.float32),
        interpret=interpret,
    )(thT, projT, dg, fbp, beta_i)


def kernel(user, item, feature_i, Bi, Gu, Gi, Bp, Tu, E, Delta_Gu, Delta_Gi):
    U, FD = Tu.shape
    gamma_u, gamma_i, beta_i, dg = _sc_gamma(
        user, item, Gu, Delta_Gu, Gi, Delta_Gi, Bi)
    TuT_flat = _tc_pad(Tu.T)
    thT = _sc_theta(user, TuT_flat, FD)
    projT, fbp, f_out = _tc_project(feature_i, E, Bp)
    xui = _tc_combine(thT, projT, dg, fbp, beta_i)
    theta_u = thT.T
    return (xui, gamma_u, gamma_i, f_out, theta_u, beta_i)


# speedup vs baseline: 1.0831x; 1.0831x over previous
import functools

import jax
import jax.numpy as jnp
from jax import lax
from jax.experimental import pallas as pl
from jax.experimental.pallas import tpu as pltpu
from jax.experimental.pallas import tpu_sc as plsc

_NC = 2
_NS = 16
_NW = _NC * _NS
_CH = 128
_UPAD = 100096


def _add_rows(a_v, b_v, rows, width):
    def body(r, carry):
        for k in range(width // 16):
            s = pl.ds(16 * k, 16)
            a_v[r, s] = a_v[r, s] + b_v[r, s]
        return carry

    lax.fori_loop(0, rows, body, 0)


def _sc_gather_body(F, FD, bpw,
                    user_h, item_h, Gu_h, DGu_h, Gi_h, DGi_h, TuT_h, Bi_h,
                    gu_out, gi_out, thT_out, be_out, dg_out,
                    idx_v, idx2_v, a_v, b_v, g_v, tht_v, be_v, dg_v,
                    sem, sem2):
    cid = lax.axis_index("c")
    sid = lax.axis_index("s")
    wid = sid * _NC + cid
    for c in range(bpw // _CH):
        base = wid * bpw + c * _CH
        pltpu.sync_copy(user_h.at[pl.ds(base, _CH)], idx_v)

        def jbody(j, carry):
            for k in range(_CH // 16):
                s = pl.ds(16 * k, 16)
                idx2_v[j, s] = idx_v[s] + j * _UPAD
            pltpu.async_copy(TuT_h.at[idx2_v.at[j]], tht_v.at[j], sem2)
            return carry

        lax.fori_loop(0, FD, jbody, 0)

        pltpu.async_copy(Gu_h.at[idx_v], g_v, sem).wait()
        pltpu.async_copy(DGu_h.at[idx_v], b_v, sem).wait()
        _add_rows(g_v, b_v, _CH, F)
        pltpu.sync_copy(g_v, gu_out.at[pl.ds(base, _CH)])

        pltpu.make_async_copy(
            Gu_h.at[pl.ds(0, FD)], tht_v, sem2).wait()
        pltpu.sync_copy(tht_v, thT_out.at[:, pl.ds(base, _CH)])

        pltpu.sync_copy(item_h.at[pl.ds(base, _CH)], idx_v)
        pltpu.async_copy(Bi_h.at[idx_v], be_v, sem2)
        pltpu.async_copy(Gi_h.at[idx_v], a_v, sem).wait()
        pltpu.async_copy(DGi_h.at[idx_v], b_v, sem).wait()
        _add_rows(a_v, b_v, _CH, F)
        pltpu.sync_copy(a_v, gi_out.at[pl.ds(base, _CH)])

        lane0 = lax.iota(jnp.int32, 16) == 0

        def dgbody(r, carry):
            acc = g_v[r, pl.ds(0, 16)] * a_v[r, pl.ds(0, 16)]
            for k in range(1, F // 16):
                s = pl.ds(16 * k, 16)
                acc = acc + g_v[r, s] * a_v[r, s]
            tot = jnp.sum(acc)
            plsc.store_scatter(dg_v, [jnp.full((16,), r, jnp.int32)],
                               jnp.full((16,), tot, jnp.float32), mask=lane0)
            return carry

        lax.fori_loop(0, _CH, dgbody, 0)
        pltpu.sync_copy(dg_v, dg_out.at[pl.ds(base, _CH)])

        pltpu.make_async_copy(Bi_h.at[pl.ds(0, _CH)], be_v, sem2).wait()
        pltpu.sync_copy(be_v, be_out.at[pl.ds(base, _CH)])


def _sc_gather(user, item, Gu, DGu, Gi, DGi, TuT_flat, Bi, FD,
               interpret=False):
    B = user.shape[0]
    F = Gu.shape[1]
    bpw = B // _NW
    mesh = plsc.VectorSubcoreMesh(
        core_axis_name="c", subcore_axis_name="s",
        num_cores=_NC, num_subcores=_NS)
    f = pl.kernel(
        functools.partial(_sc_gather_body, F, FD, bpw),
        out_type=(
            jax.ShapeDtypeStruct((B, F), jnp.float32),
            jax.ShapeDtypeStruct((B, F), jnp.float32),
            jax.ShapeDtypeStruct((FD, B), jnp.float32),
            jax.ShapeDtypeStruct((B,), jnp.float32),
            jax.ShapeDtypeStruct((B,), jnp.float32),
        ),
        mesh=mesh,
        scratch_types=(
            pltpu.VMEM((_CH,), jnp.int32),
            pltpu.VMEM((FD, _CH), jnp.int32),
            pltpu.VMEM((_CH, F), jnp.float32),
            pltpu.VMEM((_CH, F), jnp.float32),
            pltpu.VMEM((_CH, F), jnp.float32),
            pltpu.VMEM((FD, _CH), jnp.float32),
            pltpu.VMEM((_CH,), jnp.float32),
            pltpu.VMEM((_CH,), jnp.float32),
            pltpu.SemaphoreType.DMA,
            pltpu.SemaphoreType.DMA,
        ),
        compiler_params=pltpu.CompilerParams(
            use_tc_tiling_on_sc=False, needs_layout_passes=False),
        interpret=interpret,
    )
    return f(user, item, Gu, DGu, Gi, DGi, TuT_flat, Bi)


def _tc_pad_body(U, t_ref, o_ref):
    for r in range(8):
        o_ref[pl.ds(r * _UPAD, U)] = t_ref[r, :]
        o_ref[pl.ds(r * _UPAD + U, _UPAD - U)] = jnp.zeros(
            (_UPAD - U,), jnp.float32)


def _tc_pad(TuT, interpret=False):
    FD, U = TuT.shape
    return pl.pallas_call(
        functools.partial(_tc_pad_body, U),
        grid=(FD // 8,),
        in_specs=[pl.BlockSpec((8, U), lambda i: (i, 0))],
        out_specs=pl.BlockSpec((8 * _UPAD,), lambda i: (i,)),
        out_shape=jax.ShapeDtypeStruct((FD * _UPAD,), jnp.float32),
        interpret=interpret,
    )(TuT)


def _tc_project_body(f_ref, E_ref, Bp_ref, projT_ref, fbp_ref, fout_ref):
    f = f_ref[...]
    proj = jnp.dot(f, E_ref[...], preferred_element_type=jnp.float32)
    projT_ref[...] = proj.T
    fbp_ref[...] = jnp.dot(f, Bp_ref[...], preferred_element_type=jnp.float32)
    fout_ref[...] = f


def _tc_project(feature_i, E, Bp, interpret=False):
    B, NIF = feature_i.shape
    FD = E.shape[1]
    BB = 1024
    return pl.pallas_call(
        _tc_project_body,
        grid=(B // BB,),
        in_specs=[
            pl.BlockSpec((BB, NIF), lambda i: (i, 0)),
            pl.BlockSpec((NIF, FD), lambda i: (0, 0)),
            pl.BlockSpec((NIF, 1), lambda i: (0, 0)),
        ],
        out_specs=[
            pl.BlockSpec((FD, BB), lambda i: (0, i)),
            pl.BlockSpec((BB, 1), lambda i: (i, 0)),
            pl.BlockSpec((BB, NIF), lambda i: (i, 0)),
        ],
        out_shape=[
            jax.ShapeDtypeStruct((FD, B), jnp.float32),
            jax.ShapeDtypeStruct((B, 1), jnp.float32),
            jax.ShapeDtypeStruct((B, NIF), jnp.float32),
        ],
        interpret=interpret,
    )(feature_i, E, Bp)


def _tc_combine_body(thT_ref, projT_ref, dg_ref, fbp_ref, be_ref, xui_ref):
    dtT = jnp.sum(thT_ref[...] * projT_ref[...], axis=0)
    xui_ref[...] = be_ref[...] + dg_ref[...] + dtT + fbp_ref[...][:, 0]


def _tc_combine(thT, projT, dg, fbp, beta_i, interpret=False):
    FD, B = thT.shape
    BB = 4096
    return pl.pallas_call(
        _tc_combine_body,
        grid=(B // BB,),
        in_specs=[
            pl.BlockSpec((FD, BB), lambda i: (0, i)),
            pl.BlockSpec((FD, BB), lambda i: (0, i)),
            pl.BlockSpec((BB,), lambda i: (i,)),
            pl.BlockSpec((BB, 1), lambda i: (i, 0)),
            pl.BlockSpec((BB,), lambda i: (i,)),
        ],
        out_specs=pl.BlockSpec((BB,), lambda i: (i,)),
        out_shape=jax.ShapeDtypeStruct((B,), jnp.float32),
        interpret=interpret,
    )(thT, projT, dg, fbp, beta_i)


def kernel(user, item, feature_i, Bi, Gu, Gi, Bp, Tu, E, Delta_Gu, Delta_Gi):
    U, FD = Tu.shape
    TuT_flat = _tc_pad(Tu.T)
    gamma_u, gamma_i, thT, beta_i, dg = _sc_gather(
        user, item, Gu, Delta_Gu, Gi, Delta_Gi, TuT_flat, Bi, FD)
    projT, fbp, f_out = _tc_project(feature_i, E, Bp)
    xui = _tc_combine(thT, projT, dg, fbp, beta_i)
    theta_u = thT.T
    return (xui, gamma_u, gamma_i, f_out, theta_u, beta_i)
